# baseline (speedup 1.0000x reference)
import functools

import jax
import jax.numpy as jnp
from jax import lax
from jax.experimental import pallas as pl
from jax.experimental.pallas import tpu as pltpu
from jax.experimental.pallas import tpu_sc as plsc

N = 10000
E = 160000
G = 128
NP = 10240
EB = 64
EP = 163840
ROWS = EP // EB
TROWS = ROWS // 16
QB = TROWS // 4
NA = 10112
ZRA = NA // 16
BN = 512
NB = NP // BN


def _sc_agg(num_chunks, with_cnt):
    mesh = plsc.VectorSubcoreMesh(core_axis_name="c", subcore_axis_name="s")
    out_type = [jax.ShapeDtypeStruct((num_chunks * NP, 128), jnp.float32)]
    if with_cnt:
        out_type.append(jax.ShapeDtypeStruct((2 * NP, 128), jnp.float32))
    passes = num_chunks // 2

    def body(table, gidx, didx, zeros_hbm, *rest):
        if with_cnt:
            ones_hbm, out_agg, out_cnt, src_v, dst_v, gbuf, acc, *sems = rest
        else:
            out_agg, src_v, dst_v, gbuf, acc, *sems = rest
        gsems = sems
        core = lax.axis_index("c")
        sub = lax.axis_index("s")

        def gather(slot, row):
            pltpu.async_copy(table.at[src_v.at[row]], gbuf.at[slot],
                             gsems[slot])

        def wait_gather(slot, row):
            pltpu.make_async_copy(table.at[src_v.at[row]], gbuf.at[slot],
                                  gsems[slot]).wait()

        for p in range(passes):
            chunk = core * passes + p
            pltpu.sync_copy(zeros_hbm, acc.at[pl.ds(sub * ZRA, ZRA)])
            plsc.subcore_barrier()

            for h in range(4):
                base = chunk * ROWS + sub * TROWS + h * QB
                pltpu.sync_copy(gidx.at[pl.ds(base, QB)], src_v)
                pltpu.sync_copy(
                    didx.at[pl.ds(sub * TROWS + h * QB, QB)], dst_v)
                for b in range(2):
                    gather(b, b)

                @pl.loop(0, QB, step=2)
                def _(j):
                    for b in range(2):
                        jj = j + b
                        wait_gather(b, jj)
                        pltpu.sync_copy(gbuf.at[b], acc.at[dst_v.at[jj]],
                                        add=True)

                        @pl.when(jj + 2 < QB)
                        def _():
                            gather(b, jj + 2)

            plsc.subcore_barrier()
            pltpu.sync_copy(acc.at[pl.ds(sub * ZRA, ZRA)],
                            out_agg.at[pl.ds(chunk * NP + sub * ZRA, ZRA)])
            plsc.subcore_barrier()

        if with_cnt:
            pltpu.sync_copy(zeros_hbm, acc.at[pl.ds(sub * ZRA, ZRA)])
            obuf = gbuf.at[0]
            pltpu.sync_copy(ones_hbm, obuf)
            plsc.subcore_barrier()

            for h in range(2):
                pltpu.sync_copy(
                    didx.at[pl.ds(core * (ROWS // 2) + sub * 2 * QB
                                  + h * QB, QB)], src_v)

                @pl.loop(0, QB, step=8)
                def _(j):
                    for b in range(8):
                        pltpu.async_copy(obuf, acc.at[src_v.at[j + b]],
                                         gsems[0], add=True)
                    for b in range(8):
                        pltpu.make_async_copy(obuf, acc.at[src_v.at[j + b]],
                                              gsems[0]).wait()

            plsc.subcore_barrier()
            pltpu.sync_copy(acc.at[pl.ds(sub * ZRA, ZRA)],
                            out_cnt.at[pl.ds(core * NP + sub * ZRA, ZRA)])

    scratch = [
        pltpu.VMEM((QB, EB), jnp.int32),
        pltpu.VMEM((QB, EB), jnp.int32),
        pltpu.VMEM((2, EB, 128), jnp.float32),
        pltpu.VMEM_SHARED((NA, 128), jnp.float32),
    ] + [pltpu.SemaphoreType.DMA] * 2

    return functools.partial(
        pl.kernel,
        out_type=out_type,
        mesh=mesh,
        scratch_types=scratch,
    )(body)


_sc_l1 = _sc_agg(2, with_cnt=True)
_sc_l2 = _sc_agg(4, with_cnt=False)


def _t1_body(x_ref, agg_ref, cnt_ref, wl_ref, wr_ref, b_ref, h_ref, cm_ref):
    cnt = jnp.maximum(cnt_ref[0] + cnt_ref[1], 1.0)
    mean = jnp.concatenate([agg_ref[0] / cnt, agg_ref[1] / cnt], axis=1)
    h = jnp.dot(mean, wl_ref[...], preferred_element_type=jnp.float32)
    h = h + jnp.dot(x_ref[...], wr_ref[...],
                    preferred_element_type=jnp.float32)
    h = jnp.maximum(h + b_ref[...], 0.0)
    rowid = (lax.broadcasted_iota(jnp.int32, (BN, 1), 0)
             + pl.program_id(0) * BN)
    h = jnp.where(rowid < N, h, 0.0)
    for c in range(4):
        h_ref[c] = h[:, c * 128:(c + 1) * 128]
    cm_ref[...] = cnt


def _t2_body(h1_ref, agg_ref, cnt_ref, bid_ref, wl_ref, wr_ref, b_ref,
             wd_ref, bd_ref, sc_ref, pool_ref, pacc, gacc):
    i = pl.program_id(0)

    @pl.when(i == 0)
    def _():
        pacc[...] = jnp.zeros_like(pacc)
        gacc[...] = jnp.zeros_like(gacc)

    cnt = cnt_ref[...]
    mean = jnp.concatenate([agg_ref[c] / cnt for c in range(4)], axis=1)
    h1 = jnp.concatenate([h1_ref[c] for c in range(4)], axis=1)
    h2 = jnp.dot(mean, wl_ref[...], preferred_element_type=jnp.float32)
    h2 = h2 + jnp.dot(h1, wr_ref[...], preferred_element_type=jnp.float32)
    h2 = jnp.maximum(h2 + b_ref[...], 0.0)
    rowid = lax.broadcasted_iota(jnp.int32, (BN, 1), 0) + i * BN
    h2 = jnp.where(rowid < N, h2, 0.0)

    bid = bid_ref[0]
    oh = (lax.broadcasted_iota(jnp.int32, (G, BN), 0) == bid
          ).astype(jnp.float32)
    pacc[...] += jnp.dot(oh, h2, preferred_element_type=jnp.float32)
    gacc[...] += jnp.dot(oh, jnp.ones((BN, 128), jnp.float32),
                         preferred_element_type=jnp.float32)

    @pl.when(i == NB - 1)
    def _():
        gc = jnp.maximum(gacc[...], 1.0)
        pooled = pacc[...] / jnp.concatenate([gc] * 4, axis=1)
        pool_ref[...] = pooled
        sc_ref[...] = (jnp.dot(pooled, wd_ref[...],
                               preferred_element_type=jnp.float32)
                       + bd_ref[...])


_t1_call = pl.pallas_call(
    _t1_body,
    grid=(NB,),
    in_specs=[
        pl.BlockSpec((BN, 256), lambda i: (i, 0)),
        pl.BlockSpec((2, BN, 128), lambda i: (0, i, 0)),
        pl.BlockSpec((2, BN, 128), lambda i: (0, i, 0)),
        pl.BlockSpec((256, 512), lambda i: (0, 0)),
        pl.BlockSpec((256, 512), lambda i: (0, 0)),
        pl.BlockSpec((1, 512), lambda i: (0, 0)),
    ],
    out_specs=[
        pl.BlockSpec((4, BN, 128), lambda i: (0, i, 0)),
        pl.BlockSpec((BN, 128), lambda i: (i, 0)),
    ],
    out_shape=[
        jax.ShapeDtypeStruct((4, NP, 128), jnp.float32),
        jax.ShapeDtypeStruct((NP, 128), jnp.float32),
    ],
)

_t2_call = pl.pallas_call(
    _t2_body,
    grid=(NB,),
    in_specs=[
        pl.BlockSpec((4, BN, 128), lambda i: (0, i, 0)),
        pl.BlockSpec((4, BN, 128), lambda i: (0, i, 0)),
        pl.BlockSpec((BN, 128), lambda i: (i, 0)),
        pl.BlockSpec((1, 1, BN), lambda i: (i, 0, 0)),
        pl.BlockSpec((512, 512), lambda i: (0, 0)),
        pl.BlockSpec((512, 512), lambda i: (0, 0)),
        pl.BlockSpec((1, 512), lambda i: (0, 0)),
        pl.BlockSpec((512, 256), lambda i: (0, 0)),
        pl.BlockSpec((1, 256), lambda i: (0, 0)),
    ],
    out_specs=[
        pl.BlockSpec((G, 256), lambda i: (0, 0)),
        pl.BlockSpec((G, 512), lambda i: (0, 0)),
    ],
    out_shape=[
        jax.ShapeDtypeStruct((G, 256), jnp.float32),
        jax.ShapeDtypeStruct((G, 512), jnp.float32),
    ],
    scratch_shapes=[
        pltpu.VMEM((G, 512), jnp.float32),
        pltpu.VMEM((G, 128), jnp.float32),
    ],
)


def kernel(x, edge_index, batch, Wl1, Wr1, b1, Wl2, Wr2, b2, Wd, bd):
    src = edge_index[0]
    dst = edge_index[1]

    x_pad = jnp.pad(x, ((0, NP - N), (0, 0)))
    xc = x_pad.reshape(NP, 2, 128).transpose(1, 0, 2).reshape(2 * NP, 128)

    src_p = jnp.pad(src, (0, EP - E))
    dst_p = jnp.pad(dst, (0, EP - E), constant_values=N)
    didx = dst_p.reshape(ROWS, EB)
    offs2 = (jnp.arange(2, dtype=jnp.int32) * NP)[:, None]
    offs4 = (jnp.arange(4, dtype=jnp.int32) * NP)[:, None]
    gidx1 = (src_p[None, :] + offs2).reshape(2 * ROWS, EB)
    gidx2 = (src_p[None, :] + offs4).reshape(4 * ROWS, EB)

    zeros_hbm = jnp.zeros((ZRA, 128), jnp.float32)
    ones_hbm = jnp.ones((EB, 128), jnp.float32)

    agg1f, cntf = _sc_l1(xc, gidx1, didx, zeros_hbm, ones_hbm)
    h1c, cntm = _t1_call(x_pad, agg1f.reshape(2, NP, 128),
                         cntf.reshape(2, NP, 128), Wl1.T, Wr1.T, b1[None, :])

    (agg2f,) = _sc_l2(h1c.reshape(4 * NP, 128), gidx2, didx, zeros_hbm)

    batch2d = jnp.pad(batch, (0, NP - N), constant_values=G).reshape(NB, 1, BN)
    scores, pooled = _t2_call(h1c, agg2f.reshape(4, NP, 128), cntm, batch2d,
                              Wl2.T, Wr2.T, b2[None, :], Wd.T, bd[None, :])
    return (scores, pooled)


# speedup vs baseline: 1.0059x; 1.0059x over previous
import functools

import jax
import jax.numpy as jnp
from jax import lax
from jax.experimental import pallas as pl
from jax.experimental.pallas import tpu as pltpu
from jax.experimental.pallas import tpu_sc as plsc

N = 10000
E = 160000
G = 128
NP = 10240
EB = 64
EP = 163840
ROWS = EP // EB
TROWS = ROWS // 16
QB = TROWS // 4
NA = 10112
ZRA = NA // 16
BN = 512
NB = NP // BN


def _sc_agg(num_chunks, with_cnt):
    mesh = plsc.VectorSubcoreMesh(core_axis_name="c", subcore_axis_name="s")
    out_type = [jax.ShapeDtypeStruct((num_chunks * NP, 128), jnp.float32)]
    if with_cnt:
        out_type.append(jax.ShapeDtypeStruct((2 * NP, 128), jnp.float32))
    passes = num_chunks // 2

    def body(table, gidx, didx, zeros_hbm, *rest):
        if with_cnt:
            ones_hbm, out_agg, out_cnt, src_v, dst_v, gbuf, acc, *sems = rest
        else:
            out_agg, src_v, dst_v, gbuf, acc, *sems = rest
        gsems = sems
        core = lax.axis_index("c")
        sub = lax.axis_index("s")

        def gather(slot, row):
            pltpu.async_copy(table.at[src_v.at[row]], gbuf.at[slot],
                             gsems[slot])

        def wait_gather(slot, row):
            pltpu.make_async_copy(table.at[src_v.at[row]], gbuf.at[slot],
                                  gsems[slot]).wait()

        pltpu.sync_copy(zeros_hbm, acc.at[pl.ds(sub * ZRA, ZRA)])
        plsc.subcore_barrier()

        for p in range(passes):
            chunk = core * passes + p

            for h in range(4):
                base = chunk * ROWS + sub * TROWS + h * QB
                pltpu.sync_copy(gidx.at[pl.ds(base, QB)], src_v)
                pltpu.sync_copy(
                    didx.at[pl.ds(sub * TROWS + h * QB, QB)], dst_v)
                for b in range(2):
                    gather(b, b)

                @pl.loop(0, QB, step=2)
                def _(j):
                    for b in range(2):
                        jj = j + b
                        wait_gather(b, jj)
                        pltpu.sync_copy(gbuf.at[b], acc.at[dst_v.at[jj]],
                                        add=True)

                        @pl.when(jj + 2 < QB)
                        def _():
                            gather(b, jj + 2)

            plsc.subcore_barrier()
            pltpu.sync_copy(acc.at[pl.ds(sub * ZRA, ZRA)],
                            out_agg.at[pl.ds(chunk * NP + sub * ZRA, ZRA)])
            if p < passes - 1 or with_cnt:
                pltpu.sync_copy(zeros_hbm, acc.at[pl.ds(sub * ZRA, ZRA)])
            plsc.subcore_barrier()

        if with_cnt:
            obuf = gbuf.at[0]
            pltpu.sync_copy(ones_hbm, obuf)

            for h in range(2):
                pltpu.sync_copy(
                    didx.at[pl.ds(core * (ROWS // 2) + sub * 2 * QB
                                  + h * QB, QB)], src_v)

                @pl.loop(0, QB, step=8)
                def _(j):
                    for b in range(8):
                        pltpu.async_copy(obuf, acc.at[src_v.at[j + b]],
                                         gsems[0], add=True)
                    for b in range(8):
                        pltpu.make_async_copy(obuf, acc.at[src_v.at[j + b]],
                                              gsems[0]).wait()

            plsc.subcore_barrier()
            pltpu.sync_copy(acc.at[pl.ds(sub * ZRA, ZRA)],
                            out_cnt.at[pl.ds(core * NP + sub * ZRA, ZRA)])

    scratch = [
        pltpu.VMEM((QB, EB), jnp.int32),
        pltpu.VMEM((QB, EB), jnp.int32),
        pltpu.VMEM((2, EB, 128), jnp.float32),
        pltpu.VMEM_SHARED((NA, 128), jnp.float32),
    ] + [pltpu.SemaphoreType.DMA] * 2

    return functools.partial(
        pl.kernel,
        out_type=out_type,
        mesh=mesh,
        scratch_types=scratch,
    )(body)


_sc_l1 = _sc_agg(2, with_cnt=True)
_sc_l2 = _sc_agg(4, with_cnt=False)


def _t1_body(x_ref, agg_ref, cnt_ref, wl_ref, wr_ref, b_ref, h_ref, cm_ref):
    cnt = jnp.maximum(cnt_ref[0] + cnt_ref[1], 1.0)
    mean = jnp.concatenate([agg_ref[0] / cnt, agg_ref[1] / cnt], axis=1)
    h = jnp.dot(mean, wl_ref[...], preferred_element_type=jnp.float32)
    h = h + jnp.dot(x_ref[...], wr_ref[...],
                    preferred_element_type=jnp.float32)
    h = jnp.maximum(h + b_ref[...], 0.0)
    rowid = (lax.broadcasted_iota(jnp.int32, (BN, 1), 0)
             + pl.program_id(0) * BN)
    h = jnp.where(rowid < N, h, 0.0)
    for c in range(4):
        h_ref[c] = h[:, c * 128:(c + 1) * 128]
    cm_ref[...] = cnt


def _t2_body(h1_ref, agg_ref, cnt_ref, bid_ref, wl_ref, wr_ref, b_ref,
             wd_ref, bd_ref, sc_ref, pool_ref, pacc, gacc):
    i = pl.program_id(0)

    @pl.when(i == 0)
    def _():
        pacc[...] = jnp.zeros_like(pacc)
        gacc[...] = jnp.zeros_like(gacc)

    cnt = cnt_ref[...]
    mean = jnp.concatenate([agg_ref[c] / cnt for c in range(4)], axis=1)
    h1 = jnp.concatenate([h1_ref[c] for c in range(4)], axis=1)
    h2 = jnp.dot(mean, wl_ref[...], preferred_element_type=jnp.float32)
    h2 = h2 + jnp.dot(h1, wr_ref[...], preferred_element_type=jnp.float32)
    h2 = jnp.maximum(h2 + b_ref[...], 0.0)
    rowid = lax.broadcasted_iota(jnp.int32, (BN, 1), 0) + i * BN
    h2 = jnp.where(rowid < N, h2, 0.0)

    bid = bid_ref[0]
    oh = (lax.broadcasted_iota(jnp.int32, (G, BN), 0) == bid
          ).astype(jnp.float32)
    pacc[...] += jnp.dot(oh, h2, preferred_element_type=jnp.float32)
    gacc[...] += jnp.dot(oh, jnp.ones((BN, 128), jnp.float32),
                         preferred_element_type=jnp.float32)

    @pl.when(i == NB - 1)
    def _():
        gc = jnp.maximum(gacc[...], 1.0)
        pooled = pacc[...] / jnp.concatenate([gc] * 4, axis=1)
        pool_ref[...] = pooled
        sc_ref[...] = (jnp.dot(pooled, wd_ref[...],
                               preferred_element_type=jnp.float32)
                       + bd_ref[...])


_t1_call = pl.pallas_call(
    _t1_body,
    grid=(NB,),
    in_specs=[
        pl.BlockSpec((BN, 256), lambda i: (i, 0)),
        pl.BlockSpec((2, BN, 128), lambda i: (0, i, 0)),
        pl.BlockSpec((2, BN, 128), lambda i: (0, i, 0)),
        pl.BlockSpec((256, 512), lambda i: (0, 0)),
        pl.BlockSpec((256, 512), lambda i: (0, 0)),
        pl.BlockSpec((1, 512), lambda i: (0, 0)),
    ],
    out_specs=[
        pl.BlockSpec((4, BN, 128), lambda i: (0, i, 0)),
        pl.BlockSpec((BN, 128), lambda i: (i, 0)),
    ],
    out_shape=[
        jax.ShapeDtypeStruct((4, NP, 128), jnp.float32),
        jax.ShapeDtypeStruct((NP, 128), jnp.float32),
    ],
)

_t2_call = pl.pallas_call(
    _t2_body,
    grid=(NB,),
    in_specs=[
        pl.BlockSpec((4, BN, 128), lambda i: (0, i, 0)),
        pl.BlockSpec((4, BN, 128), lambda i: (0, i, 0)),
        pl.BlockSpec((BN, 128), lambda i: (i, 0)),
        pl.BlockSpec((1, 1, BN), lambda i: (i, 0, 0)),
        pl.BlockSpec((512, 512), lambda i: (0, 0)),
        pl.BlockSpec((512, 512), lambda i: (0, 0)),
        pl.BlockSpec((1, 512), lambda i: (0, 0)),
        pl.BlockSpec((512, 256), lambda i: (0, 0)),
        pl.BlockSpec((1, 256), lambda i: (0, 0)),
    ],
    out_specs=[
        pl.BlockSpec((G, 256), lambda i: (0, 0)),
        pl.BlockSpec((G, 512), lambda i: (0, 0)),
    ],
    out_shape=[
        jax.ShapeDtypeStruct((G, 256), jnp.float32),
        jax.ShapeDtypeStruct((G, 512), jnp.float32),
    ],
    scratch_shapes=[
        pltpu.VMEM((G, 512), jnp.float32),
        pltpu.VMEM((G, 128), jnp.float32),
    ],
)


def kernel(x, edge_index, batch, Wl1, Wr1, b1, Wl2, Wr2, b2, Wd, bd):
    src = edge_index[0]
    dst = edge_index[1]

    x_pad = jnp.pad(x, ((0, NP - N), (0, 0)))
    xc = x_pad.reshape(NP, 2, 128).transpose(1, 0, 2).reshape(2 * NP, 128)

    src_p = jnp.pad(src, (0, EP - E))
    dst_p = jnp.pad(dst, (0, EP - E), constant_values=N)
    didx = dst_p.reshape(ROWS, EB)
    offs2 = (jnp.arange(2, dtype=jnp.int32) * NP)[:, None]
    offs4 = (jnp.arange(4, dtype=jnp.int32) * NP)[:, None]
    gidx1 = (src_p[None, :] + offs2).reshape(2 * ROWS, EB)
    gidx2 = (src_p[None, :] + offs4).reshape(4 * ROWS, EB)

    zeros_hbm = jnp.zeros((ZRA, 128), jnp.float32)
    ones_hbm = jnp.ones((EB, 128), jnp.float32)

    agg1f, cntf = _sc_l1(xc, gidx1, didx, zeros_hbm, ones_hbm)
    h1c, cntm = _t1_call(x_pad, agg1f.reshape(2, NP, 128),
                         cntf.reshape(2, NP, 128), Wl1.T, Wr1.T, b1[None, :])

    (agg2f,) = _sc_l2(h1c.reshape(4 * NP, 128), gidx2, didx, zeros_hbm)

    batch2d = jnp.pad(batch, (0, NP - N), constant_values=G).reshape(NB, 1, BN)
    scores, pooled = _t2_call(h1c, agg2f.reshape(4, NP, 128), cntm, batch2d,
                              Wl2.T, Wr2.T, b2[None, :], Wd.T, bd[None, :])
    return (scores, pooled)
